# baseline (speedup 1.0000x reference)
import functools

import numpy as np
import jax
import jax.numpy as jnp
from jax import lax
from jax.experimental import pallas as pl
from jax.experimental.pallas import tpu as pltpu
from jax.experimental.pallas import tpu_sc as plsc

N = 4096
H = 768
DS = 32
DG = 32
FM = 64
E = 8
FF = 1536
RES_SCALE = float(1.0 / np.sqrt(8.0))

TR = 1024
NTR = N // TR
TF = 512
NTF = N // TF + E - 1
P = NTF * TF

NC = 2
NS = 16
NW = NC * NS
TOK_W = N // NW


def _gelu(x):
    return 0.5 * x * (1.0 + lax.erf(x * float(1.0 / np.sqrt(2.0))))


def _router_body(h_ref, te_ref, lng_ref, lnb_ref, wg_ref, bg_ref,
                 wf_ref, bf_ref, wr_ref, br_ref,
                 idx_out, coef_out, rank_out, po_out, texp_out, tpt_out, lb_out,
                 utri_ref, counts_sc, imp_sc):
    t = pl.program_id(0)

    @pl.when(t == 0)
    def _init():
        counts_sc[...] = jnp.zeros_like(counts_sc)
        imp_sc[...] = jnp.zeros_like(imp_sc)
        t_r = lax.broadcasted_iota(jnp.int32, (TR, TR), 0)
        t_c = lax.broadcasted_iota(jnp.int32, (TR, TR), 1)
        utri_ref[...] = (t_r < t_c).astype(jnp.float32)

    x = h_ref[...]
    mu = jnp.mean(x, axis=-1, keepdims=True)
    var = jnp.mean(x * x, axis=-1, keepdims=True) - mu * mu
    xn = (x - mu) * lax.rsqrt(var + 1e-5)
    h_ln = xn * lng_ref[...] + lnb_ref[...]
    g = _gelu(jnp.dot(h_ln, wg_ref[...], preferred_element_type=jnp.float32)
              + bg_ref[...])
    cond = jnp.concatenate([te_ref[...], g], axis=-1)
    u = _gelu(jnp.dot(cond, wf_ref[...], preferred_element_type=jnp.float32)
              + bf_ref[...])
    ut = u.T
    lt = (jnp.dot(wr_ref[...].T, ut, preferred_element_type=jnp.float32)
          + br_ref[...].reshape(E, 1))

    m = jnp.max(lt, axis=0, keepdims=True)
    p = jnp.exp(lt - m)
    probs = p / jnp.sum(p, axis=0, keepdims=True)

    eids = lax.broadcasted_iota(jnp.int32, (E, TR), 0)
    is_max = lt >= m
    top = jnp.min(jnp.where(is_max, eids, E), axis=0)
    top_prob = jnp.max(probs, axis=0)

    onehot = (eids == top[None, :]).astype(jnp.float32)
    cum = jnp.dot(onehot, utri_ref[...], preferred_element_type=jnp.float32)
    rank_f = jnp.sum(onehot * (cum + counts_sc[...]), axis=0)
    rank = rank_f.astype(jnp.int32)

    counts_sc[...] = counts_sc[...] + jnp.sum(onehot, axis=1, keepdims=True)
    imp_sc[...] = imp_sc[...] + jnp.sum(probs, axis=1, keepdims=True)

    idx_out[...] = top
    coef_out[...] = RES_SCALE * top_prob
    rank_out[...] = rank

    @pl.when(t == NTR - 1)
    def _final():
        cnt = counts_sc[...]
        pc = jnp.ceil(cnt / TF) * TF
        e_r = lax.broadcasted_iota(jnp.int32, (E, E), 0)
        e_c = lax.broadcasted_iota(jnp.int32, (E, E), 1)
        l8 = (e_c < e_r).astype(jnp.float32)
        po = jnp.dot(l8, pc, preferred_element_type=jnp.float32)
        tp = jnp.sum(pc)
        ends = po + pc
        s = lax.broadcasted_iota(jnp.int32, (1, NTF), 1).astype(jnp.float32) * TF
        sp = jnp.minimum(s, tp - TF)
        te = jnp.sum((ends <= sp).astype(jnp.int32), axis=0)
        po_out[...] = po.astype(jnp.int32).reshape(E)
        texp_out[...] = te
        tpt_out[...] = (tp.astype(jnp.int32) // TF).reshape(1)
        lb = E * jnp.sum(imp_sc[...] * cnt) / (N * N + 1e-8)
        lb_out[...] = lb.reshape(1)


def _router_call(h, tok_emb, ln_g, ln_b, Wg, bg, Wf, bf, Wr, br):
    out_shapes = (
        jax.ShapeDtypeStruct((N,), jnp.int32),
        jax.ShapeDtypeStruct((N,), jnp.float32),
        jax.ShapeDtypeStruct((N,), jnp.int32),
        jax.ShapeDtypeStruct((E,), jnp.int32),
        jax.ShapeDtypeStruct((NTF,), jnp.int32),
        jax.ShapeDtypeStruct((1,), jnp.int32),
        jax.ShapeDtypeStruct((1,), jnp.float32),
    )
    grid = (NTR,)
    tile1 = lambda i: (i,)
    const1 = lambda i: (0,)
    const2 = lambda i: (0, 0)
    in_specs = [
        pl.BlockSpec((TR, H), lambda i: (i, 0)),
        pl.BlockSpec((TR, DS), lambda i: (i, 0)),
        pl.BlockSpec((1, H), const2),
        pl.BlockSpec((1, H), const2),
        pl.BlockSpec((H, DG), const2),
        pl.BlockSpec((1, DG), const2),
        pl.BlockSpec((DS + DG, FM), const2),
        pl.BlockSpec((1, FM), const2),
        pl.BlockSpec((FM, E), const2),
        pl.BlockSpec((1, E), const2),
    ]
    out_specs = (
        pl.BlockSpec((TR,), tile1),
        pl.BlockSpec((TR,), tile1),
        pl.BlockSpec((TR,), tile1),
        pl.BlockSpec((E,), const1),
        pl.BlockSpec((NTF,), const1),
        pl.BlockSpec((1,), const1),
        pl.BlockSpec((1,), const1),
    )
    return pl.pallas_call(
        _router_body,
        grid=grid,
        in_specs=in_specs,
        out_specs=out_specs,
        out_shape=out_shapes,
        scratch_shapes=[
            pltpu.VMEM((TR, TR), jnp.float32),
            pltpu.VMEM((E, 1), jnp.float32),
            pltpu.VMEM((E, 1), jnp.float32),
        ],
        compiler_params=pltpu.CompilerParams(
            dimension_semantics=("arbitrary",)),
    )(h, tok_emb, ln_g.reshape(1, H),
      ln_b.reshape(1, H), Wg, bg.reshape(1, DG), Wf, bf.reshape(1, FM),
      Wr, br.reshape(1, E))


WROW = H + 128


NCH = 4
CHW = TOK_W // NCH


def _dispatch_body(h_hbm, idx_hbm, rank_hbm, po_hbm, coef_hbm,
                   xs_hbm, dest_hbm,
                   idx_v, rank_v, po_v, dest2_v, coef_v, rows_l, rsem_l, wsem):
    wid = lax.axis_index("s") * NC + lax.axis_index("c")
    base = wid * TOK_W
    pltpu.sync_copy(idx_hbm.at[pl.ds(base, TOK_W)], idx_v)
    pltpu.sync_copy(rank_hbm.at[pl.ds(base, TOK_W)], rank_v)
    pltpu.sync_copy(po_hbm, po_v)
    pltpu.sync_copy(coef_hbm.at[pl.ds(base, TOK_W)], coef_v)
    reads = [
        pltpu.async_copy(h_hbm.at[pl.ds(base + c * CHW, CHW)],
                         rows_l[c].at[:, pl.ds(0, H)], rsem_l[c])
        for c in range(NCH)
    ]
    for c in range(NCH):
        for j in range(CHW // 16):
            k = c * CHW + j * 16
            e16 = idx_v[pl.ds(k, 16)]
            r16 = rank_v[pl.ds(k, 16)]
            off = plsc.load_gather(po_v, [e16])
            dest2_v[c, pl.ds(j * 16, 16)] = off + r16

    for cc in range(NCH):
        def rep_body(r, carry, cc=cc):
            s16 = plsc.load_gather(coef_v,
                                   [jnp.zeros((16,), jnp.int32) + cc * CHW + r])
            rows_l[cc][r, pl.ds(H, 16)] = s16
            return carry

        lax.fori_loop(0, CHW, rep_body, 0)
    scats = []
    for c in range(NCH):
        reads[c].wait()
        scats.append(pltpu.async_copy(rows_l[c], xs_hbm.at[dest2_v.at[c]],
                                      wsem))
        pltpu.sync_copy(dest2_v.at[c], dest_hbm.at[pl.ds(base + c * CHW, CHW)])
    for s in scats:
        s.wait()


def _dispatch_call(h, top_idx, rank, po, coef):
    mesh = plsc.VectorSubcoreMesh(core_axis_name="c", subcore_axis_name="s",
                                  num_cores=NC, num_subcores=NS)
    f = pl.kernel(
        _dispatch_body,
        out_type=(
            jax.ShapeDtypeStruct((P, WROW), jnp.float32),
            jax.ShapeDtypeStruct((N,), jnp.int32),
        ),
        mesh=mesh,
        scratch_types=[
            pltpu.VMEM((TOK_W,), jnp.int32),
            pltpu.VMEM((TOK_W,), jnp.int32),
            pltpu.VMEM((E,), jnp.int32),
            pltpu.VMEM((NCH, CHW), jnp.int32),
            pltpu.VMEM((TOK_W,), jnp.float32),
            [pltpu.VMEM((CHW, WROW), jnp.float32) for _ in range(NCH)],
            [pltpu.SemaphoreType.DMA for _ in range(NCH)],
            pltpu.SemaphoreType.DMA,
        ],
        compiler_params=pltpu.CompilerParams(needs_layout_passes=False),
    )
    return f(h, top_idx, rank, po, coef)


def _ffn_body(te_ref, tpt_ref, x_ref, w1_ref, b1_ref, w2_ref, b2_ref,
              y_ref, w1b, w2b, preve):
    t = pl.program_id(0)

    @pl.when(t < tpt_ref[0])
    def _():
        x = x_ref[:, :H]
        cf = x_ref[:, H:H + 1]
        e = te_ref[t]
        hmid = _gelu(jnp.dot(x, w1_ref[0],
                             preferred_element_type=jnp.float32)
                     + b1_ref[pl.ds(e, 1), :])
        y = (jnp.dot(hmid, w2_ref[0],
                     preferred_element_type=jnp.float32)
             + b2_ref[pl.ds(e, 1), :])
        y_ref[...] = x + cf * y


def _ffn_call(xs, W1, b1, W2, b2, te, tpt):
    def xmap(t, te_ref, tpt_ref):
        return (jnp.minimum(t, tpt_ref[0] - 1), 0)

    def emap3(t, te_ref, tpt_ref):
        return (te_ref[t], 0, 0)

    grid_spec = pltpu.PrefetchScalarGridSpec(
        num_scalar_prefetch=2,
        grid=(NTF,),
        in_specs=[
            pl.BlockSpec((TF, WROW), xmap),
            pl.BlockSpec((1, H, FF), emap3),
            pl.BlockSpec((E, FF), lambda t, te_ref, tpt_ref: (0, 0)),
            pl.BlockSpec((1, FF, H), emap3),
            pl.BlockSpec((E, H), lambda t, te_ref, tpt_ref: (0, 0)),
        ],
        out_specs=pl.BlockSpec((TF, H), xmap),
        scratch_shapes=[
            pltpu.VMEM((H, FF), jnp.bfloat16),
            pltpu.VMEM((FF, H), jnp.bfloat16),
            pltpu.SMEM((1,), jnp.int32),
        ],
    )
    return pl.pallas_call(
        _ffn_body,
        grid_spec=grid_spec,
        out_shape=jax.ShapeDtypeStruct((P, H), jnp.float32),
        compiler_params=pltpu.CompilerParams(
            dimension_semantics=("arbitrary",)),
    )(te, tpt, xs, W1, b1, W2, b2)


def _combine_body(ys_hbm, dest_hbm, out_hbm, dest2_v, yv_l, gsem_l, wsem):
    wid = lax.axis_index("s") * NC + lax.axis_index("c")
    base = wid * TOK_W
    for c in range(NCH):
        pltpu.sync_copy(dest_hbm.at[pl.ds(base + c * CHW, CHW)],
                        dest2_v.at[c])
    gets = [
        pltpu.async_copy(ys_hbm.at[dest2_v.at[c]], yv_l[c], gsem_l[c])
        for c in range(NCH)
    ]
    writes = []
    for c in range(NCH):
        gets[c].wait()
        writes.append(pltpu.async_copy(
            yv_l[c], out_hbm.at[pl.ds(base + c * CHW, CHW)], wsem))
    for w in writes:
        w.wait()


def _combine_call(ys, dest):
    mesh = plsc.VectorSubcoreMesh(core_axis_name="c", subcore_axis_name="s",
                                  num_cores=NC, num_subcores=NS)
    f = pl.kernel(
        _combine_body,
        out_type=jax.ShapeDtypeStruct((N, H), jnp.float32),
        mesh=mesh,
        scratch_types=[
            pltpu.VMEM((NCH, CHW), jnp.int32),
            [pltpu.VMEM((CHW, H), jnp.float32) for _ in range(NCH)],
            [pltpu.SemaphoreType.DMA for _ in range(NCH)],
            pltpu.SemaphoreType.DMA,
        ],
        compiler_params=pltpu.CompilerParams(needs_layout_passes=False),
    )
    return f(ys, dest)


def kernel(h, tok_emb, is_mask, ln_g, ln_b, Wg, bg, Wf, bf, Wr, br, W1, b1, W2, b2):
    top_idx, coef, rank, po, te, tpt, lb = _router_call(
        h, tok_emb, ln_g, ln_b, Wg, bg, Wf, bf, Wr, br)
    xs, dest = _dispatch_call(h, top_idx, rank, po, coef)
    ys = _ffn_call(xs, W1, b1, W2, b2, te, tpt)
    h_out = _combine_call(ys, dest)
    return (h_out, lb.reshape(()))


# speedup vs baseline: 1.0125x; 1.0125x over previous
import functools

import numpy as np
import jax
import jax.numpy as jnp
from jax import lax
from jax.experimental import pallas as pl
from jax.experimental.pallas import tpu as pltpu
from jax.experimental.pallas import tpu_sc as plsc

N = 4096
H = 768
DS = 32
DG = 32
FM = 64
E = 8
FF = 1536
RES_SCALE = float(1.0 / np.sqrt(8.0))

TR = 1024
NTR = N // TR
TF = 512
NTF = N // TF + E - 1
P = NTF * TF

NC = 2
NS = 16
NW = NC * NS
TOK_W = N // NW


def _gelu(x):
    return 0.5 * x * (1.0 + lax.erf(x * float(1.0 / np.sqrt(2.0))))


def _router_body(h_ref, te_ref, lng_ref, lnb_ref, wg_ref, bg_ref,
                 wf_ref, bf_ref, wr_ref, br_ref,
                 idx_out, coef_out, rank_out, po_out, texp_out, tpt_out, lb_out,
                 utri_ref, counts_sc, imp_sc):
    t = pl.program_id(0)

    @pl.when(t == 0)
    def _init():
        counts_sc[...] = jnp.zeros_like(counts_sc)
        imp_sc[...] = jnp.zeros_like(imp_sc)
        t_r = lax.broadcasted_iota(jnp.int32, (TR, TR), 0)
        t_c = lax.broadcasted_iota(jnp.int32, (TR, TR), 1)
        utri_ref[...] = (t_r < t_c).astype(jnp.float32)

    x = h_ref[...]
    mu = jnp.mean(x, axis=-1, keepdims=True)
    var = jnp.mean(x * x, axis=-1, keepdims=True) - mu * mu
    xn = (x - mu) * lax.rsqrt(var + 1e-5)
    h_ln = xn * lng_ref[...] + lnb_ref[...]
    g = _gelu(jnp.dot(h_ln, wg_ref[...], preferred_element_type=jnp.float32)
              + bg_ref[...])
    cond = jnp.concatenate([te_ref[...], g], axis=-1)
    u = _gelu(jnp.dot(cond, wf_ref[...], preferred_element_type=jnp.float32)
              + bf_ref[...])
    ut = u.T
    lt = (jnp.dot(wr_ref[...].T, ut, preferred_element_type=jnp.float32)
          + br_ref[...].reshape(E, 1))

    m = jnp.max(lt, axis=0, keepdims=True)
    p = jnp.exp(lt - m)
    probs = p / jnp.sum(p, axis=0, keepdims=True)

    eids = lax.broadcasted_iota(jnp.int32, (E, TR), 0)
    is_max = lt >= m
    top = jnp.min(jnp.where(is_max, eids, E), axis=0)
    top_prob = jnp.max(probs, axis=0)

    onehot = (eids == top[None, :]).astype(jnp.float32)
    cum = jnp.dot(onehot, utri_ref[...], preferred_element_type=jnp.float32)
    rank_f = jnp.sum(onehot * (cum + counts_sc[...]), axis=0)
    rank = rank_f.astype(jnp.int32)

    counts_sc[...] = counts_sc[...] + jnp.sum(onehot, axis=1, keepdims=True)
    imp_sc[...] = imp_sc[...] + jnp.sum(probs, axis=1, keepdims=True)

    idx_out[...] = top
    coef_out[...] = RES_SCALE * top_prob
    rank_out[...] = rank

    @pl.when(t == NTR - 1)
    def _final():
        cnt = counts_sc[...]
        pc = jnp.ceil(cnt / TF) * TF
        e_r = lax.broadcasted_iota(jnp.int32, (E, E), 0)
        e_c = lax.broadcasted_iota(jnp.int32, (E, E), 1)
        l8 = (e_c < e_r).astype(jnp.float32)
        po = jnp.dot(l8, pc, preferred_element_type=jnp.float32)
        tp = jnp.sum(pc)
        ends = po + pc
        s = lax.broadcasted_iota(jnp.int32, (1, NTF), 1).astype(jnp.float32) * TF
        sp = jnp.minimum(s, tp - TF)
        te = jnp.sum((ends <= sp).astype(jnp.int32), axis=0)
        po_out[...] = po.astype(jnp.int32).reshape(E)
        texp_out[...] = te
        tpt_out[...] = (tp.astype(jnp.int32) // TF).reshape(1)
        lb = E * jnp.sum(imp_sc[...] * cnt) / (N * N + 1e-8)
        lb_out[...] = lb.reshape(1)


def _router_call(h, tok_emb, ln_g, ln_b, Wg, bg, Wf, bf, Wr, br):
    out_shapes = (
        jax.ShapeDtypeStruct((N,), jnp.int32),
        jax.ShapeDtypeStruct((N,), jnp.float32),
        jax.ShapeDtypeStruct((N,), jnp.int32),
        jax.ShapeDtypeStruct((E,), jnp.int32),
        jax.ShapeDtypeStruct((NTF,), jnp.int32),
        jax.ShapeDtypeStruct((1,), jnp.int32),
        jax.ShapeDtypeStruct((1,), jnp.float32),
    )
    grid = (NTR,)
    tile1 = lambda i: (i,)
    const1 = lambda i: (0,)
    const2 = lambda i: (0, 0)
    in_specs = [
        pl.BlockSpec((TR, H), lambda i: (i, 0)),
        pl.BlockSpec((TR, DS), lambda i: (i, 0)),
        pl.BlockSpec((1, H), const2),
        pl.BlockSpec((1, H), const2),
        pl.BlockSpec((H, DG), const2),
        pl.BlockSpec((1, DG), const2),
        pl.BlockSpec((DS + DG, FM), const2),
        pl.BlockSpec((1, FM), const2),
        pl.BlockSpec((FM, E), const2),
        pl.BlockSpec((1, E), const2),
    ]
    out_specs = (
        pl.BlockSpec((TR,), tile1),
        pl.BlockSpec((TR,), tile1),
        pl.BlockSpec((TR,), tile1),
        pl.BlockSpec((E,), const1),
        pl.BlockSpec((NTF,), const1),
        pl.BlockSpec((1,), const1),
        pl.BlockSpec((1,), const1),
    )
    return pl.pallas_call(
        _router_body,
        grid=grid,
        in_specs=in_specs,
        out_specs=out_specs,
        out_shape=out_shapes,
        scratch_shapes=[
            pltpu.VMEM((TR, TR), jnp.float32),
            pltpu.VMEM((E, 1), jnp.float32),
            pltpu.VMEM((E, 1), jnp.float32),
        ],
        compiler_params=pltpu.CompilerParams(
            dimension_semantics=("arbitrary",)),
    )(h, tok_emb, ln_g.reshape(1, H),
      ln_b.reshape(1, H), Wg, bg.reshape(1, DG), Wf, bf.reshape(1, FM),
      Wr, br.reshape(1, E))


WROW = H + 128


def _dispatch_body(h_hbm, idx_hbm, rank_hbm, po_hbm, coef_hbm,
                   xs_hbm, dest_hbm,
                   idx_v, rank_v, po_v, dest_v, coef_v, rows_v,
                   sem):
    wid = lax.axis_index("s") * NC + lax.axis_index("c")
    base = wid * TOK_W
    pltpu.sync_copy(idx_hbm.at[pl.ds(base, TOK_W)], idx_v)
    pltpu.sync_copy(rank_hbm.at[pl.ds(base, TOK_W)], rank_v)
    pltpu.sync_copy(po_hbm, po_v)
    pltpu.sync_copy(coef_hbm.at[pl.ds(base, TOK_W)], coef_v)
    for j in range(TOK_W // 16):
        e16 = idx_v[pl.ds(j * 16, 16)]
        r16 = rank_v[pl.ds(j * 16, 16)]
        off = plsc.load_gather(po_v, [e16])
        dest_v[pl.ds(j * 16, 16)] = off + r16

    def rep_body(r, carry):
        s16 = plsc.load_gather(coef_v, [jnp.zeros((16,), jnp.int32) + r])
        rows_v[r, pl.ds(H, 16)] = s16
        return carry

    lax.fori_loop(0, TOK_W, rep_body, 0)
    pltpu.sync_copy(h_hbm.at[pl.ds(base, TOK_W)], rows_v.at[:, pl.ds(0, H)])
    pltpu.async_copy(rows_v, xs_hbm.at[dest_v], sem).wait()
    pltpu.sync_copy(dest_v, dest_hbm.at[pl.ds(base, TOK_W)])


def _dispatch_call(h, top_idx, rank, po, coef):
    mesh = plsc.VectorSubcoreMesh(core_axis_name="c", subcore_axis_name="s",
                                  num_cores=NC, num_subcores=NS)
    f = pl.kernel(
        _dispatch_body,
        out_type=(
            jax.ShapeDtypeStruct((P, WROW), jnp.float32),
            jax.ShapeDtypeStruct((N,), jnp.int32),
        ),
        mesh=mesh,
        scratch_types=[
            pltpu.VMEM((TOK_W,), jnp.int32),
            pltpu.VMEM((TOK_W,), jnp.int32),
            pltpu.VMEM((E,), jnp.int32),
            pltpu.VMEM((TOK_W,), jnp.int32),
            pltpu.VMEM((TOK_W,), jnp.float32),
            pltpu.VMEM((TOK_W, WROW), jnp.float32),
            pltpu.SemaphoreType.DMA,
        ],
        compiler_params=pltpu.CompilerParams(needs_layout_passes=False),
    )
    return f(h, top_idx, rank, po, coef)


def _ffn_body(te_ref, tpt_ref, x_ref, w1_ref, b1_ref, w2_ref, b2_ref,
              y_ref, w1b, w2b, preve):
    t = pl.program_id(0)

    @pl.when(t < tpt_ref[0])
    def _():
        x = x_ref[:, :H]
        cf = x_ref[:, H:H + 1]
        e = te_ref[t]
        hmid = _gelu(jnp.dot(x, w1_ref[0],
                             preferred_element_type=jnp.float32)
                     + b1_ref[pl.ds(e, 1), :])
        y = (jnp.dot(hmid, w2_ref[0],
                     preferred_element_type=jnp.float32)
             + b2_ref[pl.ds(e, 1), :])
        y_ref[...] = x + cf * y


def _ffn_call(xs, W1, b1, W2, b2, te, tpt):
    def xmap(t, te_ref, tpt_ref):
        return (jnp.minimum(t, tpt_ref[0] - 1), 0)

    def emap3(t, te_ref, tpt_ref):
        return (te_ref[t], 0, 0)

    grid_spec = pltpu.PrefetchScalarGridSpec(
        num_scalar_prefetch=2,
        grid=(NTF,),
        in_specs=[
            pl.BlockSpec((TF, WROW), xmap),
            pl.BlockSpec((1, H, FF), emap3),
            pl.BlockSpec((E, FF), lambda t, te_ref, tpt_ref: (0, 0)),
            pl.BlockSpec((1, FF, H), emap3),
            pl.BlockSpec((E, H), lambda t, te_ref, tpt_ref: (0, 0)),
        ],
        out_specs=pl.BlockSpec((TF, H), xmap),
        scratch_shapes=[
            pltpu.VMEM((H, FF), jnp.bfloat16),
            pltpu.VMEM((FF, H), jnp.bfloat16),
            pltpu.SMEM((1,), jnp.int32),
        ],
    )
    return pl.pallas_call(
        _ffn_body,
        grid_spec=grid_spec,
        out_shape=jax.ShapeDtypeStruct((P, H), jnp.float32),
        compiler_params=pltpu.CompilerParams(
            dimension_semantics=("arbitrary",)),
    )(te, tpt, xs, W1, b1, W2, b2)


def _combine_body(ys_hbm, dest_hbm, out_hbm, destc_v, yv, sem):
    wid = lax.axis_index("s") * NC + lax.axis_index("c")
    base = wid * TOK_W
    pltpu.sync_copy(dest_hbm.at[pl.ds(base, TOK_W)], destc_v)
    pltpu.async_copy(ys_hbm.at[destc_v], yv, sem).wait()
    pltpu.sync_copy(yv, out_hbm.at[pl.ds(base, TOK_W)])


def _combine_call(ys, dest):
    mesh = plsc.VectorSubcoreMesh(core_axis_name="c", subcore_axis_name="s",
                                  num_cores=NC, num_subcores=NS)
    f = pl.kernel(
        _combine_body,
        out_type=jax.ShapeDtypeStruct((N, H), jnp.float32),
        mesh=mesh,
        scratch_types=[
            pltpu.VMEM((TOK_W,), jnp.int32),
            pltpu.VMEM((TOK_W, H), jnp.float32),
            pltpu.SemaphoreType.DMA,
        ],
        compiler_params=pltpu.CompilerParams(needs_layout_passes=False),
    )
    return f(ys, dest)


def kernel(h, tok_emb, is_mask, ln_g, ln_b, Wg, bg, Wf, bf, Wr, br, W1, b1, W2, b2):
    top_idx, coef, rank, po, te, tpt, lb = _router_call(
        h, tok_emb, ln_g, ln_b, Wg, bg, Wf, bf, Wr, br)
    xs, dest = _dispatch_call(h, top_idx, rank, po, coef)
    ys = _ffn_call(xs, W1, b1, W2, b2, te, tpt)
    h_out = _combine_call(ys, dest)
    return (h_out, lb.reshape(()))


# speedup vs baseline: 1.0145x; 1.0019x over previous
import numpy as np
import jax
import jax.numpy as jnp
from jax import lax
from jax.experimental import pallas as pl
from jax.experimental.pallas import tpu as pltpu
from jax.experimental.pallas import tpu_sc as plsc

N = 4096
H = 768
DS = 32
DG = 32
FM = 64
E = 8
FF = 1536
RES_SCALE = float(1.0 / np.sqrt(8.0))

TR = 1024
NTR = N // TR
TF = 512
NTF = N // TF + E - 1
P = NTF * TF

NC = 2
NS = 16
NW = NC * NS
TOK_W = N // NW


def _gelu(x):
    return 0.5 * x * (1.0 + lax.erf(x * float(1.0 / np.sqrt(2.0))))


def _router_body(h_ref, te_ref, lng_ref, lnb_ref, wg_ref, bg_ref,
                 wf_ref, bf_ref, wr_ref, br_ref,
                 idx_out, coef_out, rank_out, po_out, texp_out, tpt_out, lb_out,
                 utri_ref, counts_sc, imp_sc):
    t = pl.program_id(0)

    @pl.when(t == 0)
    def _init():
        counts_sc[...] = jnp.zeros_like(counts_sc)
        imp_sc[...] = jnp.zeros_like(imp_sc)
        t_r = lax.broadcasted_iota(jnp.int32, (TR, TR), 0)
        t_c = lax.broadcasted_iota(jnp.int32, (TR, TR), 1)
        utri_ref[...] = (t_r < t_c).astype(jnp.float32)

    x = h_ref[...]
    mu = jnp.mean(x, axis=-1, keepdims=True)
    var = jnp.mean(x * x, axis=-1, keepdims=True) - mu * mu
    xn = (x - mu) * lax.rsqrt(var + 1e-5)
    h_ln = xn * lng_ref[...] + lnb_ref[...]
    g = _gelu(jnp.dot(h_ln, wg_ref[...], preferred_element_type=jnp.float32)
              + bg_ref[...])
    cond = jnp.concatenate([te_ref[...], g], axis=-1)
    u = _gelu(jnp.dot(cond, wf_ref[...], preferred_element_type=jnp.float32)
              + bf_ref[...])
    ut = u.T
    lt = (jnp.dot(wr_ref[...].T, ut, preferred_element_type=jnp.float32)
          + br_ref[...].reshape(E, 1))

    m = jnp.max(lt, axis=0, keepdims=True)
    p = jnp.exp(lt - m)
    probs = p / jnp.sum(p, axis=0, keepdims=True)

    eids = lax.broadcasted_iota(jnp.int32, (E, TR), 0)
    is_max = lt >= m
    top = jnp.min(jnp.where(is_max, eids, E), axis=0)
    top_prob = jnp.max(probs, axis=0)

    onehot = (eids == top[None, :]).astype(jnp.float32)
    cum = jnp.dot(onehot, utri_ref[...], preferred_element_type=jnp.float32)
    rank_f = jnp.sum(onehot * (cum + counts_sc[...]), axis=0)
    rank = rank_f.astype(jnp.int32)

    counts_sc[...] = counts_sc[...] + jnp.sum(onehot, axis=1, keepdims=True)
    imp_sc[...] = imp_sc[...] + jnp.sum(probs, axis=1, keepdims=True)

    idx_out[...] = top
    coef_out[...] = RES_SCALE * top_prob
    rank_out[...] = rank

    @pl.when(t == NTR - 1)
    def _final():
        cnt = counts_sc[...]
        pc = jnp.ceil(cnt / TF) * TF
        e_r = lax.broadcasted_iota(jnp.int32, (E, E), 0)
        e_c = lax.broadcasted_iota(jnp.int32, (E, E), 1)
        l8 = (e_c < e_r).astype(jnp.float32)
        po = jnp.dot(l8, pc, preferred_element_type=jnp.float32)
        tp = jnp.sum(pc)
        ends = po + pc
        s = lax.broadcasted_iota(jnp.int32, (1, NTF), 1).astype(jnp.float32) * TF
        sp = jnp.minimum(s, tp - TF)
        te = jnp.sum((ends <= sp).astype(jnp.int32), axis=0)
        po_out[...] = po.astype(jnp.int32).reshape(E)
        texp_out[...] = te
        tpt_out[...] = (tp.astype(jnp.int32) // TF).reshape(1)
        lb = E * jnp.sum(imp_sc[...] * cnt) / (N * N + 1e-8)
        lb_out[...] = lb.reshape(1)


def _router_call(h, tok_emb, ln_g, ln_b, Wg, bg, Wf, bf, Wr, br):
    out_shapes = (
        jax.ShapeDtypeStruct((N,), jnp.int32),
        jax.ShapeDtypeStruct((N,), jnp.float32),
        jax.ShapeDtypeStruct((N,), jnp.int32),
        jax.ShapeDtypeStruct((E,), jnp.int32),
        jax.ShapeDtypeStruct((NTF,), jnp.int32),
        jax.ShapeDtypeStruct((1,), jnp.int32),
        jax.ShapeDtypeStruct((1,), jnp.float32),
    )
    grid = (NTR,)
    tile1 = lambda i: (i,)
    const1 = lambda i: (0,)
    const2 = lambda i: (0, 0)
    in_specs = [
        pl.BlockSpec((TR, H), lambda i: (i, 0)),
        pl.BlockSpec((TR, DS), lambda i: (i, 0)),
        pl.BlockSpec((1, H), const2),
        pl.BlockSpec((1, H), const2),
        pl.BlockSpec((H, DG), const2),
        pl.BlockSpec((1, DG), const2),
        pl.BlockSpec((DS + DG, FM), const2),
        pl.BlockSpec((1, FM), const2),
        pl.BlockSpec((FM, E), const2),
        pl.BlockSpec((1, E), const2),
    ]
    out_specs = (
        pl.BlockSpec((TR,), tile1),
        pl.BlockSpec((TR,), tile1),
        pl.BlockSpec((TR,), tile1),
        pl.BlockSpec((E,), const1),
        pl.BlockSpec((NTF,), const1),
        pl.BlockSpec((1,), const1),
        pl.BlockSpec((1,), const1),
    )
    return pl.pallas_call(
        _router_body,
        grid=grid,
        in_specs=in_specs,
        out_specs=out_specs,
        out_shape=out_shapes,
        scratch_shapes=[
            pltpu.VMEM((TR, TR), jnp.float32),
            pltpu.VMEM((E, 1), jnp.float32),
            pltpu.VMEM((E, 1), jnp.float32),
        ],
        compiler_params=pltpu.CompilerParams(
            dimension_semantics=("arbitrary",)),
    )(h, tok_emb, ln_g.reshape(1, H),
      ln_b.reshape(1, H), Wg, bg.reshape(1, DG), Wf, bf.reshape(1, FM),
      Wr, br.reshape(1, E))


WROW = H + 128


def _dispatch_body(h_hbm, idx_hbm, rank_hbm, po_hbm, coef_hbm,
                   xs_hbm, dest_hbm,
                   idx_v, rank_v, po_v, dest_v, coef_v, rows_v,
                   sem):
    wid = lax.axis_index("s") * NC + lax.axis_index("c")
    base = wid * TOK_W
    pltpu.sync_copy(idx_hbm.at[pl.ds(base, TOK_W)], idx_v)
    pltpu.sync_copy(rank_hbm.at[pl.ds(base, TOK_W)], rank_v)
    pltpu.sync_copy(po_hbm, po_v)
    pltpu.sync_copy(coef_hbm.at[pl.ds(base, TOK_W)], coef_v)
    for j in range(TOK_W // 16):
        e16 = idx_v[pl.ds(j * 16, 16)]
        r16 = rank_v[pl.ds(j * 16, 16)]
        off = plsc.load_gather(po_v, [e16])
        dest_v[pl.ds(j * 16, 16)] = off + r16

    def rep_body(r, carry):
        s16 = plsc.load_gather(coef_v, [jnp.zeros((16,), jnp.int32) + r])
        rows_v[r, pl.ds(H, 16)] = s16
        return carry

    lax.fori_loop(0, TOK_W, rep_body, 0)
    pltpu.sync_copy(h_hbm.at[pl.ds(base, TOK_W)], rows_v.at[:, pl.ds(0, H)])
    pltpu.async_copy(rows_v, xs_hbm.at[dest_v], sem).wait()
    pltpu.sync_copy(dest_v, dest_hbm.at[pl.ds(base, TOK_W)])


def _dispatch_call(h, top_idx, rank, po, coef):
    mesh = plsc.VectorSubcoreMesh(core_axis_name="c", subcore_axis_name="s",
                                  num_cores=NC, num_subcores=NS)
    f = pl.kernel(
        _dispatch_body,
        out_type=(
            jax.ShapeDtypeStruct((P, WROW), jnp.float32),
            jax.ShapeDtypeStruct((N,), jnp.int32),
        ),
        mesh=mesh,
        scratch_types=[
            pltpu.VMEM((TOK_W,), jnp.int32),
            pltpu.VMEM((TOK_W,), jnp.int32),
            pltpu.VMEM((E,), jnp.int32),
            pltpu.VMEM((TOK_W,), jnp.int32),
            pltpu.VMEM((TOK_W,), jnp.float32),
            pltpu.VMEM((TOK_W, WROW), jnp.float32),
            pltpu.SemaphoreType.DMA,
        ],
        compiler_params=pltpu.CompilerParams(needs_layout_passes=False),
    )
    return f(h, top_idx, rank, po, coef)


def _ffn_body(te_ref, tpt_ref, x_ref, w1_ref, b1_ref, w2_ref, b2_ref,
              y_ref, w1b, w2b, preve):
    t = pl.program_id(0)

    @pl.when(t < tpt_ref[0])
    def _():
        x = x_ref[:, :H]
        cf = x_ref[:, H:H + 1]
        e = te_ref[t]
        hmid = _gelu(jnp.dot(x, w1_ref[0],
                             preferred_element_type=jnp.float32)
                     + b1_ref[pl.ds(e, 1), :])
        y = (jnp.dot(hmid, w2_ref[0],
                     preferred_element_type=jnp.float32)
             + b2_ref[pl.ds(e, 1), :])
        y_ref[...] = x + cf * y


def _ffn_call(xs, W1, b1, W2, b2, te, tpt):
    def xmap(t, te_ref, tpt_ref):
        return (jnp.minimum(t, tpt_ref[0] - 1), 0)

    def emap3(t, te_ref, tpt_ref):
        return (te_ref[t], 0, 0)

    grid_spec = pltpu.PrefetchScalarGridSpec(
        num_scalar_prefetch=2,
        grid=(NTF,),
        in_specs=[
            pl.BlockSpec((TF, WROW), xmap),
            pl.BlockSpec((1, H, FF), emap3),
            pl.BlockSpec((E, FF), lambda t, te_ref, tpt_ref: (0, 0)),
            pl.BlockSpec((1, FF, H), emap3),
            pl.BlockSpec((E, H), lambda t, te_ref, tpt_ref: (0, 0)),
        ],
        out_specs=pl.BlockSpec((TF, H), xmap),
        scratch_shapes=[
            pltpu.VMEM((H, FF), jnp.bfloat16),
            pltpu.VMEM((FF, H), jnp.bfloat16),
            pltpu.SMEM((1,), jnp.int32),
        ],
    )
    return pl.pallas_call(
        _ffn_body,
        grid_spec=grid_spec,
        out_shape=jax.ShapeDtypeStruct((P, H), jnp.float32),
        compiler_params=pltpu.CompilerParams(
            dimension_semantics=("arbitrary",)),
    )(te, tpt, xs, W1, b1, W2, b2)


def _combine_body(ys_hbm, dest_hbm, out_hbm, destc_v, yv, sem):
    wid = lax.axis_index("s") * NC + lax.axis_index("c")
    base = wid * TOK_W
    pltpu.sync_copy(dest_hbm.at[pl.ds(base, TOK_W)], destc_v)
    pltpu.async_copy(ys_hbm.at[destc_v], yv, sem).wait()
    pltpu.sync_copy(yv, out_hbm.at[pl.ds(base, TOK_W)])


def _combine_call(ys, dest):
    mesh = plsc.VectorSubcoreMesh(core_axis_name="c", subcore_axis_name="s",
                                  num_cores=NC, num_subcores=NS)
    f = pl.kernel(
        _combine_body,
        out_type=jax.ShapeDtypeStruct((N, H), jnp.float32),
        mesh=mesh,
        scratch_types=[
            pltpu.VMEM((TOK_W,), jnp.int32),
            pltpu.VMEM((TOK_W, H), jnp.float32),
            pltpu.SemaphoreType.DMA,
        ],
        compiler_params=pltpu.CompilerParams(needs_layout_passes=False),
    )
    return f(ys, dest)


def kernel(h, tok_emb, is_mask, ln_g, ln_b, Wg, bg, Wf, bf, Wr, br, W1, b1, W2, b2):
    top_idx, coef, rank, po, te, tpt, lb = _router_call(
        h, tok_emb, ln_g, ln_b, Wg, bg, Wf, bf, Wr, br)
    xs, dest = _dispatch_call(h, top_idx, rank, po, coef)
    ys = _ffn_call(xs, W1, b1, W2, b2, te, tpt)
    h_out = _combine_call(ys, dest)
    return (h_out, lb.reshape(()))
